# Initial kernel scaffold; baseline (speedup 1.0000x reference)
import jax
import jax.numpy as jnp
from jax.experimental import pallas as pl


def kernel(x, group_features):
    raise NotImplementedError("write your pallas kernel here")


# speedup vs baseline: 1.7032x; 1.7032x over previous
import functools

import jax
import jax.numpy as jnp
from jax.experimental import pallas as pl
from jax.experimental.pallas import tpu as pltpu

_N_GROUPS = 8192
_DIM = 256
_BETA = 0.99
_TOKENS = 65536
_TM = 256


def _assign_accum_body(x_ref, gf_ref, sums_ref, counts_ref, rnorm_ref):
    i = pl.program_id(0)

    @pl.when(i == 0)
    def _init():
        gf = gf_ref[...]
        ns = jax.lax.dot_general(
            jnp.ones((1, _DIM), jnp.float32), gf * gf,
            (((1,), (1,)), ((), ())), preferred_element_type=jnp.float32)
        rnorm_ref[...] = 1.0 / jnp.maximum(jnp.sqrt(ns), 1e-12)
        sums_ref[...] = jnp.zeros_like(sums_ref)
        counts_ref[...] = jnp.zeros_like(counts_ref)

    x = x_ref[...]
    logits = jax.lax.dot_general(
        x, gf_ref[...], (((1,), (1,)), ((), ())),
        preferred_element_type=jnp.float32)
    logits = logits * rnorm_ref[...]
    assign = jnp.argmax(logits, axis=1, keepdims=True).astype(jnp.int32)
    gid = jax.lax.broadcasted_iota(jnp.int32, (_TM, _N_GROUPS), 1)
    onehot = (gid == assign).astype(jnp.float32)
    sums_ref[...] += jax.lax.dot_general(
        onehot, x, (((0,), (0,)), ((), ())),
        preferred_element_type=jnp.float32)
    counts_ref[...] += jnp.sum(onehot, axis=0, keepdims=True)


def _blend_body(gf_ref, sums_ref, cnt_ref, out_ref):
    r = 1.0 / jnp.maximum(cnt_ref[...], 1.0)
    out_ref[...] = _BETA * gf_ref[...] + (1.0 - _BETA) * sums_ref[...] * r


@jax.jit
def kernel(x, group_features):
    grid = _TOKENS // _TM
    sums, counts = pl.pallas_call(
        _assign_accum_body,
        grid=(grid,),
        in_specs=[
            pl.BlockSpec((_TM, _DIM), lambda i: (i, 0)),
            pl.BlockSpec((_N_GROUPS, _DIM), lambda i: (0, 0)),
        ],
        out_specs=[
            pl.BlockSpec((_N_GROUPS, _DIM), lambda i: (0, 0)),
            pl.BlockSpec((1, _N_GROUPS), lambda i: (0, 0)),
        ],
        out_shape=[
            jax.ShapeDtypeStruct((_N_GROUPS, _DIM), jnp.float32),
            jax.ShapeDtypeStruct((1, _N_GROUPS), jnp.float32),
        ],
        scratch_shapes=[pltpu.VMEM((1, _N_GROUPS), jnp.float32)],
        compiler_params=pltpu.CompilerParams(
            dimension_semantics=("arbitrary",)),
    )(x, group_features)

    counts_col = counts.reshape(_N_GROUPS, 1)
    rows = 1024
    out = pl.pallas_call(
        _blend_body,
        grid=(_N_GROUPS // rows,),
        in_specs=[
            pl.BlockSpec((rows, _DIM), lambda i: (i, 0)),
            pl.BlockSpec((rows, _DIM), lambda i: (i, 0)),
            pl.BlockSpec((rows, 1), lambda i: (i, 0)),
        ],
        out_specs=pl.BlockSpec((rows, _DIM), lambda i: (i, 0)),
        out_shape=jax.ShapeDtypeStruct((_N_GROUPS, _DIM), jnp.float32),
    )(group_features, sums, counts_col)
    return out


# speedup vs baseline: 2.2312x; 1.3100x over previous
import functools

import jax
import jax.numpy as jnp
from jax.experimental import pallas as pl
from jax.experimental.pallas import tpu as pltpu

_N_GROUPS = 8192
_DIM = 256
_BETA = 0.99
_TOKENS = 65536
_TM = 256


def _assign_accum_body(x_ref, gf_ref, sums_ref, counts_ref, gfn_ref):
    i = pl.program_id(0)

    @pl.when(i == 0)
    def _init():
        gf = gf_ref[...]
        ns = jnp.sum(gf * gf, axis=1, keepdims=True)
        rnorm = 1.0 / jnp.maximum(jnp.sqrt(ns), 1e-12)
        gfn_ref[...] = (gf * rnorm).astype(jnp.bfloat16)
        sums_ref[...] = jnp.zeros_like(sums_ref)
        counts_ref[...] = jnp.zeros_like(counts_ref)

    x = x_ref[...].astype(jnp.bfloat16)
    logits = jax.lax.dot_general(
        x, gfn_ref[...], (((1,), (1,)), ((), ())),
        preferred_element_type=jnp.float32)
    assign = jnp.argmax(logits, axis=1, keepdims=True).astype(jnp.int32)
    gid = jax.lax.broadcasted_iota(jnp.int32, (_TM, _N_GROUPS), 1)
    onehot = (gid == assign).astype(jnp.bfloat16)
    sums_ref[...] += jax.lax.dot_general(
        onehot, x, (((0,), (0,)), ((), ())),
        preferred_element_type=jnp.float32)
    counts_ref[...] += jnp.sum(onehot.astype(jnp.float32), axis=0,
                               keepdims=True)


def _blend_body(gf_ref, sums_ref, cnt_ref, out_ref):
    r = 1.0 / jnp.maximum(cnt_ref[...], 1.0)
    out_ref[...] = _BETA * gf_ref[...] + (1.0 - _BETA) * sums_ref[...] * r


@jax.jit
def kernel(x, group_features):
    grid = _TOKENS // _TM
    sums, counts = pl.pallas_call(
        _assign_accum_body,
        grid=(grid,),
        in_specs=[
            pl.BlockSpec((_TM, _DIM), lambda i: (i, 0)),
            pl.BlockSpec((_N_GROUPS, _DIM), lambda i: (0, 0)),
        ],
        out_specs=[
            pl.BlockSpec((_N_GROUPS, _DIM), lambda i: (0, 0)),
            pl.BlockSpec((1, _N_GROUPS), lambda i: (0, 0)),
        ],
        out_shape=[
            jax.ShapeDtypeStruct((_N_GROUPS, _DIM), jnp.float32),
            jax.ShapeDtypeStruct((1, _N_GROUPS), jnp.float32),
        ],
        scratch_shapes=[pltpu.VMEM((_N_GROUPS, _DIM), jnp.bfloat16)],
        compiler_params=pltpu.CompilerParams(
            dimension_semantics=("arbitrary",)),
    )(x, group_features)

    counts_col = counts.reshape(_N_GROUPS, 1)
    rows = 1024
    out = pl.pallas_call(
        _blend_body,
        grid=(_N_GROUPS // rows,),
        in_specs=[
            pl.BlockSpec((rows, _DIM), lambda i: (i, 0)),
            pl.BlockSpec((rows, _DIM), lambda i: (i, 0)),
            pl.BlockSpec((rows, 1), lambda i: (i, 0)),
        ],
        out_specs=pl.BlockSpec((rows, _DIM), lambda i: (i, 0)),
        out_shape=jax.ShapeDtypeStruct((_N_GROUPS, _DIM), jnp.float32),
    )(group_features, sums, counts_col)
    return out
